# baseline (speedup 1.0000x reference)
import functools

import jax
import jax.numpy as jnp
from jax import lax
from jax.experimental import pallas as pl
from jax.experimental.pallas import tpu as pltpu
from jax.experimental.pallas import tpu_sc as plsc

B = 4
S = 2048
HID = 768
LANES = 16
HVECS = HID // LANES

_info = plsc.get_sparse_core_info()
NC = _info.num_cores
NS = _info.num_subcores
NW = NC * NS

TOKENS = B * S
TPW = TOKENS // NW
CHUNK = 32
NCHUNK = TPW // CHUNK
WORKERS_PER_BATCH = S // TPW


def _body(ids_hbm, tt_hbm, w_word, w_pos, w_type, out_hbm,
          idx_v, tt_v, acc_v, wbuf_v, tbuf_v, sem):
    wid = lax.axis_index("s") * NC + lax.axis_index("c")

    pltpu.sync_copy(ids_hbm.at[pl.ds(wid * NCHUNK, NCHUNK)], idx_v)
    pltpu.sync_copy(tt_hbm.at[pl.ds(wid * NCHUNK, NCHUNK)], tt_v)

    def accum(i, _):
        for k in range(HVECS):
            sl = pl.ds(k * LANES, LANES)
            plsc.addupdate(acc_v.at[i, sl], wbuf_v[i, sl] + tbuf_v[i, sl])
        return _

    for c in range(NCHUNK):
        base = wid * TPW + c * CHUNK
        p0 = (wid % WORKERS_PER_BATCH) * TPW + c * CHUNK
        pltpu.sync_copy(w_pos.at[pl.ds(p0, CHUNK)], acc_v)
        pltpu.async_copy(w_word.at[idx_v.at[c]], wbuf_v, sem).wait()
        pltpu.async_copy(w_type.at[tt_v.at[c]], tbuf_v, sem).wait()
        lax.fori_loop(0, CHUNK, accum, 0, unroll=False)
        pltpu.sync_copy(acc_v, out_hbm.at[pl.ds(base, CHUNK)])


def kernel(input_ids, token_type_ids, W_word, W_pos, W_type):
    ids = input_ids.reshape(TOKENS).astype(jnp.int32).reshape(NW * NCHUNK, CHUNK)
    tts = token_type_ids.reshape(TOKENS).astype(jnp.int32).reshape(NW * NCHUNK, CHUNK)

    mesh = plsc.VectorSubcoreMesh(core_axis_name="c", subcore_axis_name="s")
    out = pl.kernel(
        _body,
        out_type=jax.ShapeDtypeStruct((TOKENS, HID), jnp.float32),
        mesh=mesh,
        scratch_types=[
            pltpu.VMEM((NCHUNK, CHUNK), jnp.int32),
            pltpu.VMEM((NCHUNK, CHUNK), jnp.int32),
            pltpu.VMEM((CHUNK, HID), jnp.float32),
            pltpu.VMEM((CHUNK, HID), jnp.float32),
            pltpu.VMEM((CHUNK, HID), jnp.float32),
            pltpu.SemaphoreType.DMA,
        ],
    )(ids, tts, W_word, W_pos, W_type)
    return out.reshape(B, S, HID)


# speedup vs baseline: 1.0506x; 1.0506x over previous
import jax
import jax.numpy as jnp
from jax import lax
from jax.experimental import pallas as pl
from jax.experimental.pallas import tpu as pltpu
from jax.experimental.pallas import tpu_sc as plsc

B = 4
S = 2048
HID = 768
LANES = 16
HVECS = HID // LANES

_info = plsc.get_sparse_core_info()
NC = _info.num_cores
NS = _info.num_subcores
NW = NC * NS

TOKENS = B * S
POS_PER_W = S // NW
CHUNK = 8
SUBS = POS_PER_W // CHUNK
NCHUNK = SUBS * B
DEPTH = 4
AHEAD = 2


def _body(ids_hbm, tt_hbm, w_word, w_pos, w_type, out_hbm,
          idx_v, tt_v, pbuf, wbuf, tbuf,
          sem_w, sem_t, sem_o, sem_p):
    wid = lax.axis_index("s") * NC + lax.axis_index("c")

    pltpu.sync_copy(ids_hbm.at[pl.ds(wid * NCHUNK, NCHUNK)], idx_v)
    pltpu.sync_copy(tt_hbm.at[pl.ds(wid * NCHUNK, NCHUNK)], tt_v)
    pltpu.sync_copy(w_pos.at[pl.ds(wid * POS_PER_W, CHUNK)], pbuf.at[0])

    def gather_desc(c):
        q = lax.rem(c, DEPTH)
        gw = pltpu.make_async_copy(
            w_word.at[idx_v.at[c]], wbuf.at[q], sem_w.at[lax.rem(c, AHEAD)])
        gt = pltpu.make_async_copy(
            w_type.at[tt_v.at[c]], tbuf.at[q], sem_t.at[lax.rem(c, AHEAD)])
        return gw, gt

    def store_desc(c):
        q = lax.rem(c, DEPTH)
        s = lax.div(c, B)
        b = lax.rem(c, B)
        base = b * S + wid * POS_PER_W + s * CHUNK
        return pltpu.make_async_copy(
            wbuf.at[q], out_hbm.at[pl.ds(base, CHUNK)],
            sem_o.at[lax.rem(c, AHEAD)])

    def pos_desc(s):
        return pltpu.make_async_copy(
            w_pos.at[pl.ds(wid * POS_PER_W + s * CHUNK, CHUNK)],
            pbuf.at[lax.rem(s, 2)], sem_p)

    for c in range(AHEAD):
        gw, gt = gather_desc(c)
        gw.start()
        gt.start()

    def chunk_body(c, carry):
        q = lax.rem(c, DEPTH)
        s = lax.div(c, B)
        b = lax.rem(c, B)

        gw, gt = gather_desc(c)
        gw.wait()
        gt.wait()

        @pl.when(jnp.logical_and(b == 0, s > 0))
        def _():
            pos_desc(s).wait()

        @pl.when(jnp.logical_and(b == 0, s + 1 < SUBS))
        def _():
            pos_desc(s + 1).start()

        sq = lax.rem(s, 2)

        def accum(i, _):
            for k in range(HVECS):
                sl = pl.ds(k * LANES, LANES)
                plsc.addupdate(wbuf.at[q, i, sl], pbuf[sq, i, sl] + tbuf[q, i, sl])
            return _

        lax.fori_loop(0, CHUNK, accum, 0, unroll=False)

        store_desc(c).start()

        @pl.when(c >= AHEAD)
        def _():
            store_desc(c - AHEAD).wait()

        @pl.when(c + AHEAD < NCHUNK)
        def _():
            gw2, gt2 = gather_desc(c + AHEAD)
            gw2.start()
            gt2.start()

        return carry

    lax.fori_loop(0, NCHUNK, chunk_body, 0, unroll=False)

    for c in range(NCHUNK - AHEAD, NCHUNK):
        store_desc(c).wait()


def kernel(input_ids, token_type_ids, W_word, W_pos, W_type):
    def order(x):
        x = x.reshape(B, NW, SUBS, CHUNK).astype(jnp.int32)
        return x.transpose(1, 2, 0, 3).reshape(NW * NCHUNK, CHUNK)

    ids = order(input_ids)
    tts = order(token_type_ids)

    mesh = plsc.VectorSubcoreMesh(core_axis_name="c", subcore_axis_name="s")
    out = pl.kernel(
        _body,
        out_type=jax.ShapeDtypeStruct((TOKENS, HID), jnp.float32),
        mesh=mesh,
        scratch_types=[
            pltpu.VMEM((NCHUNK, CHUNK), jnp.int32),
            pltpu.VMEM((NCHUNK, CHUNK), jnp.int32),
            pltpu.VMEM((2, CHUNK, HID), jnp.float32),
            pltpu.VMEM((DEPTH, CHUNK, HID), jnp.float32),
            pltpu.VMEM((DEPTH, CHUNK, HID), jnp.float32),
            pltpu.SemaphoreType.DMA((AHEAD,)),
            pltpu.SemaphoreType.DMA((AHEAD,)),
            pltpu.SemaphoreType.DMA((AHEAD,)),
            pltpu.SemaphoreType.DMA,
        ],
    )(ids, tts, W_word, W_pos, W_type)

    return out.reshape(B, S, HID)


# speedup vs baseline: 1.0607x; 1.0096x over previous
import jax
import jax.numpy as jnp
from jax import lax
from jax.experimental import pallas as pl
from jax.experimental.pallas import tpu as pltpu
from jax.experimental.pallas import tpu_sc as plsc

B = 4
S = 2048
HID = 768
LANES = 16
HVECS = HID // LANES

_info = plsc.get_sparse_core_info()
NC = _info.num_cores
NS = _info.num_subcores
NW = NC * NS

TOKENS = B * S
POS_PER_W = S // NW
CHUNK = 8
SUBS = POS_PER_W // CHUNK
NCHUNK = SUBS * B
DEPTH = 4
AHEAD = 2


def _body(ids_hbm, tt_hbm, w_word, w_pos, w_type, out_hbm,
          idx_v, tt_v, pbuf, wbuf, tbuf,
          sem_w, sem_t, sem_o, sem_p):
    wid = lax.axis_index("s") * NC + lax.axis_index("c")

    pltpu.sync_copy(ids_hbm.at[pl.ds(wid * NCHUNK, NCHUNK)], idx_v)
    pltpu.sync_copy(tt_hbm.at[pl.ds(wid * NCHUNK, NCHUNK)], tt_v)
    pltpu.sync_copy(w_pos.at[pl.ds(wid * POS_PER_W, CHUNK)], pbuf.at[0])

    def gather_desc(c):
        q = lax.rem(c, DEPTH)
        gw = pltpu.make_async_copy(
            w_word.at[idx_v.at[c]], wbuf.at[q], sem_w.at[lax.rem(c, AHEAD)])
        gt = pltpu.make_async_copy(
            w_type.at[tt_v.at[c]], tbuf.at[q], sem_t.at[lax.rem(c, AHEAD)])
        return gw, gt

    def store_desc(c):
        q = lax.rem(c, DEPTH)
        s = lax.div(c, B)
        b = lax.rem(c, B)
        base = b * S + wid * POS_PER_W + s * CHUNK
        return pltpu.make_async_copy(
            wbuf.at[q], out_hbm.at[pl.ds(base, CHUNK)],
            sem_o.at[lax.rem(c, AHEAD)])

    def pos_desc(s):
        return pltpu.make_async_copy(
            w_pos.at[pl.ds(wid * POS_PER_W + s * CHUNK, CHUNK)],
            pbuf.at[lax.rem(s, 2)], sem_p)

    for c in range(AHEAD):
        gw, gt = gather_desc(c)
        gw.start()
        gt.start()

    def chunk_body(c, carry):
        q = lax.rem(c, DEPTH)
        s = lax.div(c, B)
        b = lax.rem(c, B)

        gw, gt = gather_desc(c)
        gw.wait()
        gt.wait()

        @pl.when(jnp.logical_and(b == 0, s > 0))
        def _():
            pos_desc(s).wait()

        @pl.when(jnp.logical_and(b == 0, s + 1 < SUBS))
        def _():
            pos_desc(s + 1).start()

        sq = lax.rem(s, 2)

        @plsc.parallel_loop(0, CHUNK * HVECS, unroll=4)
        def accum(t):
            i = lax.div(t, HVECS)
            k = lax.rem(t, HVECS)
            sl = pl.ds(k * LANES, LANES)
            plsc.addupdate(wbuf.at[q, i, sl], pbuf[sq, i, sl] + tbuf[q, i, sl])

        store_desc(c).start()

        @pl.when(c >= AHEAD)
        def _():
            store_desc(c - AHEAD).wait()

        @pl.when(c + AHEAD < NCHUNK)
        def _():
            gw2, gt2 = gather_desc(c + AHEAD)
            gw2.start()
            gt2.start()

        return carry

    lax.fori_loop(0, NCHUNK, chunk_body, 0, unroll=False)

    for c in range(NCHUNK - AHEAD, NCHUNK):
        store_desc(c).wait()


def kernel(input_ids, token_type_ids, W_word, W_pos, W_type):
    def order(x):
        x = x.reshape(B, NW, SUBS, CHUNK).astype(jnp.int32)
        return x.transpose(1, 2, 0, 3).reshape(NW * NCHUNK, CHUNK)

    ids = order(input_ids)
    tts = order(token_type_ids)

    mesh = plsc.VectorSubcoreMesh(core_axis_name="c", subcore_axis_name="s")
    out = pl.kernel(
        _body,
        out_type=jax.ShapeDtypeStruct((TOKENS, HID), jnp.float32),
        mesh=mesh,
        scratch_types=[
            pltpu.VMEM((NCHUNK, CHUNK), jnp.int32),
            pltpu.VMEM((NCHUNK, CHUNK), jnp.int32),
            pltpu.VMEM((2, CHUNK, HID), jnp.float32),
            pltpu.VMEM((DEPTH, CHUNK, HID), jnp.float32),
            pltpu.VMEM((DEPTH, CHUNK, HID), jnp.float32),
            pltpu.SemaphoreType.DMA((AHEAD,)),
            pltpu.SemaphoreType.DMA((AHEAD,)),
            pltpu.SemaphoreType.DMA((AHEAD,)),
            pltpu.SemaphoreType.DMA,
        ],
    )(ids, tts, W_word, W_pos, W_type)

    return out.reshape(B, S, HID)


# speedup vs baseline: 3.1974x; 3.0143x over previous
import jax
import jax.numpy as jnp
from jax import lax
from jax.experimental import pallas as pl
from jax.experimental.pallas import tpu as pltpu
from jax.experimental.pallas import tpu_sc as plsc

B = 4
S = 2048
HID = 768
LANES = 16
HVECS = HID // LANES

_info = plsc.get_sparse_core_info()
NC = _info.num_cores
NS = _info.num_subcores
NW = NC * NS

TOKENS = B * S
POS_PER_W = S // NW
CHUNK = 8
SUBS = POS_PER_W // CHUNK
NCHUNK = SUBS * B
DEPTH = 4
AHEAD = 2


def _body(ids_hbm, tt_hbm, w_word, w_pos, w_type, out_hbm,
          idx_v, tt_v, pbuf, wbuf, tbuf,
          sem_w, sem_t, sem_o, sem_p):
    wid = lax.axis_index("s") * NC + lax.axis_index("c")

    pltpu.sync_copy(ids_hbm.at[pl.ds(wid * NCHUNK, NCHUNK)], idx_v)
    pltpu.sync_copy(tt_hbm.at[pl.ds(wid * NCHUNK, NCHUNK)], tt_v)
    pltpu.sync_copy(w_pos.at[pl.ds(wid * POS_PER_W, CHUNK)], pbuf.at[0])

    def gather_desc(c):
        q = lax.rem(c, DEPTH)
        gw = pltpu.make_async_copy(
            w_word.at[idx_v.at[c]], wbuf.at[q], sem_w.at[lax.rem(c, AHEAD)])
        gt = pltpu.make_async_copy(
            w_type.at[tt_v.at[c]], tbuf.at[q], sem_t.at[lax.rem(c, AHEAD)])
        return gw, gt

    def store_desc(c):
        q = lax.rem(c, DEPTH)
        s = lax.div(c, B)
        b = lax.rem(c, B)
        base = b * S + wid * POS_PER_W + s * CHUNK
        return pltpu.make_async_copy(
            wbuf.at[q], out_hbm.at[pl.ds(base, CHUNK)],
            sem_o.at[lax.rem(c, AHEAD)])

    def pos_desc(s):
        return pltpu.make_async_copy(
            w_pos.at[pl.ds(wid * POS_PER_W + s * CHUNK, CHUNK)],
            pbuf.at[lax.rem(s, 2)], sem_p)

    for c in range(AHEAD):
        gw, gt = gather_desc(c)
        gw.start()
        gt.start()

    def chunk_body(c, carry):
        q = lax.rem(c, DEPTH)
        s = lax.div(c, B)
        b = lax.rem(c, B)

        gw, gt = gather_desc(c)
        gw.wait()
        gt.wait()

        @pl.when(jnp.logical_and(b == 0, s > 0))
        def _():
            pos_desc(s).wait()

        @pl.when(jnp.logical_and(b == 0, s + 1 < SUBS))
        def _():
            pos_desc(s + 1).start()

        sq = lax.rem(s, 2)

        @plsc.parallel_loop(0, CHUNK * HVECS, unroll=4)
        def accum(t):
            i = lax.div(t, HVECS)
            k = lax.rem(t, HVECS)
            sl = pl.ds(k * LANES, LANES)
            plsc.addupdate(wbuf.at[q, i, sl], pbuf[sq, i, sl] + tbuf[q, i, sl])

        store_desc(c).start()

        @pl.when(c >= AHEAD)
        def _():
            store_desc(c - AHEAD).wait()

        @pl.when(c + AHEAD < NCHUNK)
        def _():
            gw2, gt2 = gather_desc(c + AHEAD)
            gw2.start()
            gt2.start()

        return carry

    lax.fori_loop(0, NCHUNK, chunk_body, 0, unroll=False)

    for c in range(NCHUNK - AHEAD, NCHUNK):
        store_desc(c).wait()


def kernel(input_ids, token_type_ids, W_word, W_pos, W_type):
    def order(x):
        x = x.reshape(B, NW, SUBS, CHUNK).astype(jnp.int32)
        return x.transpose(1, 2, 0, 3).reshape(NW * NCHUNK, CHUNK)

    ids = order(input_ids)
    tts = order(token_type_ids)
    spread = jnp.arange(tts.size, dtype=jnp.int32).reshape(tts.shape) % 64
    tts = tts + 2 * spread
    w_type_tiled = jnp.tile(W_type, (64, 1))

    mesh = plsc.VectorSubcoreMesh(core_axis_name="c", subcore_axis_name="s")
    out = pl.kernel(
        _body,
        out_type=jax.ShapeDtypeStruct((TOKENS, HID), jnp.float32),
        mesh=mesh,
        scratch_types=[
            pltpu.VMEM((NCHUNK, CHUNK), jnp.int32),
            pltpu.VMEM((NCHUNK, CHUNK), jnp.int32),
            pltpu.VMEM((2, CHUNK, HID), jnp.float32),
            pltpu.VMEM((DEPTH, CHUNK, HID), jnp.float32),
            pltpu.VMEM((DEPTH, CHUNK, HID), jnp.float32),
            pltpu.SemaphoreType.DMA((AHEAD,)),
            pltpu.SemaphoreType.DMA((AHEAD,)),
            pltpu.SemaphoreType.DMA((AHEAD,)),
            pltpu.SemaphoreType.DMA,
        ],
    )(ids, tts, W_word, W_pos, w_type_tiled)

    return out.reshape(B, S, HID)


# speedup vs baseline: 4.7486x; 1.4851x over previous
import jax
import jax.numpy as jnp
from jax import lax
from jax.experimental import pallas as pl
from jax.experimental.pallas import tpu as pltpu
from jax.experimental.pallas import tpu_sc as plsc

B = 4
S = 2048
HID = 768
LANES = 16
HVECS = HID // LANES

_info = plsc.get_sparse_core_info()
NC = _info.num_cores
NS = _info.num_subcores
NW = NC * NS

TOKENS = B * S
POS_PER_W = S // NW
CHUNK = 16
SUBS = POS_PER_W // CHUNK
NCHUNK = SUBS * B
DEPTH = 4
AHEAD = 2


def _body(ids_hbm, ttb_hbm, w_word, w_pos, w_type, out_hbm,
          idx_v, ttb_v, type_v, pbuf, wbuf,
          sem_w, sem_b, sem_o, sem_p):
    wid = lax.axis_index("s") * NC + lax.axis_index("c")

    pltpu.sync_copy(ids_hbm.at[pl.ds(wid * NCHUNK, NCHUNK)], idx_v)
    pltpu.sync_copy(w_type, type_v)
    pltpu.sync_copy(w_pos.at[pl.ds(wid * POS_PER_W, CHUNK)], pbuf.at[0])

    def gather_desc(c):
        q = lax.rem(c, DEPTH)
        gw = pltpu.make_async_copy(
            w_word.at[idx_v.at[c]], wbuf.at[q], sem_w.at[lax.rem(c, AHEAD)])
        gb = pltpu.make_async_copy(
            ttb_hbm.at[wid * NCHUNK + c], ttb_v.at[q],
            sem_b.at[lax.rem(c, AHEAD)])
        return gw, gb

    def store_desc(c):
        q = lax.rem(c, DEPTH)
        s = lax.div(c, B)
        b = lax.rem(c, B)
        base = b * S + wid * POS_PER_W + s * CHUNK
        return pltpu.make_async_copy(
            wbuf.at[q], out_hbm.at[pl.ds(base, CHUNK)],
            sem_o.at[lax.rem(c, AHEAD)])

    def pos_desc(s):
        return pltpu.make_async_copy(
            w_pos.at[pl.ds(wid * POS_PER_W + s * CHUNK, CHUNK)],
            pbuf.at[lax.rem(s, 2)], sem_p)

    for c in range(AHEAD):
        gw, gb = gather_desc(c)
        gw.start()
        gb.start()

    def chunk_body(c, carry):
        q = lax.rem(c, DEPTH)
        s = lax.div(c, B)
        b = lax.rem(c, B)

        gw, gb = gather_desc(c)
        gw.wait()
        gb.wait()

        @pl.when(jnp.logical_and(b == 0, s > 0))
        def _():
            pos_desc(s).wait()

        @pl.when(jnp.logical_and(b == 0, s + 1 < SUBS))
        def _():
            pos_desc(s + 1).start()

        sq = lax.rem(s, 2)

        @plsc.parallel_loop(0, HVECS)
        def accum(k):
            sl = pl.ds(k * LANES, LANES)
            t0k = type_v[0, sl]
            dk = type_v[1, sl] - t0k
            for i in range(CHUNK):
                ttb = ttb_v[q, i, :]
                y = pbuf[sq, i, sl] + t0k + ttb * dk
                plsc.addupdate(wbuf.at[q, i, sl], y)

        store_desc(c).start()

        @pl.when(c >= AHEAD)
        def _():
            store_desc(c - AHEAD).wait()

        @pl.when(c + AHEAD < NCHUNK)
        def _():
            gw2, gb2 = gather_desc(c + AHEAD)
            gw2.start()
            gb2.start()

        return carry

    lax.fori_loop(0, NCHUNK, chunk_body, 0, unroll=False)

    for c in range(NCHUNK - AHEAD, NCHUNK):
        store_desc(c).wait()


def kernel(input_ids, token_type_ids, W_word, W_pos, W_type):
    def order(x):
        x = x.reshape(B, NW, SUBS, CHUNK)
        return x.transpose(1, 2, 0, 3).reshape(NW * NCHUNK, CHUNK)

    ids = order(input_ids.astype(jnp.int32))
    ttb = jnp.repeat(
        order(token_type_ids.astype(jnp.float32))[..., None], LANES, axis=-1)

    mesh = plsc.VectorSubcoreMesh(core_axis_name="c", subcore_axis_name="s")
    out = pl.kernel(
        _body,
        out_type=jax.ShapeDtypeStruct((TOKENS, HID), jnp.float32),
        mesh=mesh,
        scratch_types=[
            pltpu.VMEM((NCHUNK, CHUNK), jnp.int32),
            pltpu.VMEM((DEPTH, CHUNK, LANES), jnp.float32),
            pltpu.VMEM((2, HID), jnp.float32),
            pltpu.VMEM((2, CHUNK, HID), jnp.float32),
            pltpu.VMEM((DEPTH, CHUNK, HID), jnp.float32),
            pltpu.SemaphoreType.DMA((AHEAD,)),
            pltpu.SemaphoreType.DMA((AHEAD,)),
            pltpu.SemaphoreType.DMA((AHEAD,)),
            pltpu.SemaphoreType.DMA,
        ],
    )(ids, ttb, W_word, W_pos, W_type)

    return out.reshape(B, S, HID)


# speedup vs baseline: 4.7678x; 1.0040x over previous
import jax
import jax.numpy as jnp
from jax import lax
from jax.experimental import pallas as pl
from jax.experimental.pallas import tpu as pltpu
from jax.experimental.pallas import tpu_sc as plsc

B = 4
S = 2048
HID = 768
LANES = 16
HVECS = HID // LANES

_info = plsc.get_sparse_core_info()
NC = _info.num_cores
NS = _info.num_subcores
NW = NC * NS

TOKENS = B * S
POS_PER_W = S // NW
CHUNK = 16
SUBS = POS_PER_W // CHUNK
NCHUNK = SUBS * B
DEPTH = 5
AHEAD = 3


def _body(ids_hbm, ttb_hbm, w_word, w_pos, w_type, out_hbm,
          idx_v, ttb_v, type_v, pbuf, wbuf,
          sem_w, sem_b, sem_o, sem_p):
    wid = lax.axis_index("s") * NC + lax.axis_index("c")

    pltpu.sync_copy(ids_hbm.at[pl.ds(wid * NCHUNK, NCHUNK)], idx_v)
    pltpu.sync_copy(w_type, type_v)
    pltpu.sync_copy(w_pos.at[pl.ds(wid * POS_PER_W, CHUNK)], pbuf.at[0])

    def gather_desc(c):
        q = lax.rem(c, DEPTH)
        gw = pltpu.make_async_copy(
            w_word.at[idx_v.at[c]], wbuf.at[q], sem_w.at[lax.rem(c, AHEAD)])
        gb = pltpu.make_async_copy(
            ttb_hbm.at[wid * NCHUNK + c], ttb_v.at[q],
            sem_b.at[lax.rem(c, AHEAD)])
        return gw, gb

    def store_desc(c):
        q = lax.rem(c, DEPTH)
        s = lax.div(c, B)
        b = lax.rem(c, B)
        base = b * S + wid * POS_PER_W + s * CHUNK
        return pltpu.make_async_copy(
            wbuf.at[q], out_hbm.at[pl.ds(base, CHUNK)],
            sem_o.at[lax.rem(c, AHEAD)])

    def pos_desc(s):
        return pltpu.make_async_copy(
            w_pos.at[pl.ds(wid * POS_PER_W + s * CHUNK, CHUNK)],
            pbuf.at[lax.rem(s, 2)], sem_p)

    for c in range(AHEAD):
        gw, gb = gather_desc(c)
        gw.start()
        gb.start()

    def chunk_body(c, carry):
        q = lax.rem(c, DEPTH)
        s = lax.div(c, B)
        b = lax.rem(c, B)

        gw, gb = gather_desc(c)
        gw.wait()
        gb.wait()

        @pl.when(jnp.logical_and(b == 0, s > 0))
        def _():
            pos_desc(s).wait()

        @pl.when(jnp.logical_and(b == 0, s + 1 < SUBS))
        def _():
            pos_desc(s + 1).start()

        sq = lax.rem(s, 2)

        @plsc.parallel_loop(0, HVECS)
        def accum(k):
            sl = pl.ds(k * LANES, LANES)
            t0k = type_v[0, sl]
            dk = type_v[1, sl] - t0k
            for i in range(CHUNK):
                ttb = ttb_v[q, i, :]
                y = pbuf[sq, i, sl] + t0k + ttb * dk
                plsc.addupdate(wbuf.at[q, i, sl], y)

        store_desc(c).start()

        @pl.when(c >= AHEAD)
        def _():
            store_desc(c - AHEAD).wait()

        @pl.when(c + AHEAD < NCHUNK)
        def _():
            gw2, gb2 = gather_desc(c + AHEAD)
            gw2.start()
            gb2.start()

        return carry

    lax.fori_loop(0, NCHUNK, chunk_body, 0, unroll=False)

    for c in range(NCHUNK - AHEAD, NCHUNK):
        store_desc(c).wait()


def kernel(input_ids, token_type_ids, W_word, W_pos, W_type):
    def order(x):
        x = x.reshape(B, NW, SUBS, CHUNK)
        return x.transpose(1, 2, 0, 3).reshape(NW * NCHUNK, CHUNK)

    ids = order(input_ids.astype(jnp.int32))
    ttb = jnp.repeat(
        order(token_type_ids.astype(jnp.float32))[..., None], LANES, axis=-1)

    mesh = plsc.VectorSubcoreMesh(core_axis_name="c", subcore_axis_name="s")
    out = pl.kernel(
        _body,
        out_type=jax.ShapeDtypeStruct((TOKENS, HID), jnp.float32),
        mesh=mesh,
        scratch_types=[
            pltpu.VMEM((NCHUNK, CHUNK), jnp.int32),
            pltpu.VMEM((DEPTH, CHUNK, LANES), jnp.float32),
            pltpu.VMEM((2, HID), jnp.float32),
            pltpu.VMEM((2, CHUNK, HID), jnp.float32),
            pltpu.VMEM((DEPTH, CHUNK, HID), jnp.float32),
            pltpu.SemaphoreType.DMA((AHEAD,)),
            pltpu.SemaphoreType.DMA((AHEAD,)),
            pltpu.SemaphoreType.DMA((AHEAD,)),
            pltpu.SemaphoreType.DMA,
        ],
    )(ids, ttb, W_word, W_pos, W_type)

    return out.reshape(B, S, HID)
